# baseline (speedup 1.0000x reference)
import functools

import jax
import jax.numpy as jnp
from jax import lax
from jax.experimental import pallas as pl
from jax.experimental.pallas import tpu as pltpu
from jax.experimental.pallas import tpu_sc as plsc

N = 10000
NP = 10240
D_IN = 256
DH = 512
E = 160000
EP = 163840
PAD_NODE = N
NC, NS = 2, 16
NW = NC * NS
EPW = EP // NW
EB = 128
NB = EP // NS // EB
RPS = NP // NS
NCHUNK = DH // 128

_sc_mesh = plsc.VectorSubcoreMesh(
    core_axis_name="c", subcore_axis_name="s", num_cores=NC, num_subcores=NS)
_sc_params = pltpu.CompilerParams(needs_layout_passes=False)


@functools.partial(
    pl.kernel,
    out_type=jax.ShapeDtypeStruct((NW * NP,), jnp.float32),
    mesh=_sc_mesh,
    scratch_types=[
        pltpu.VMEM((EPW,), jnp.int32),
        pltpu.VMEM((NP,), jnp.float32),
    ],
    compiler_params=_sc_params,
)
def _sc_degree(dst_hbm, zero_hbm, degp_hbm, dstv, hist):
    cid = lax.axis_index("c")
    sid = lax.axis_index("s")
    wid = sid * NC + cid
    pltpu.sync_copy(zero_hbm, hist)
    pltpu.sync_copy(dst_hbm.at[pl.ds(wid * EPW, EPW)], dstv)
    ones = jnp.ones((16,), jnp.float32)

    def body(j, carry):
        idx = dstv[pl.ds(j * 16, 16)]
        plsc.addupdate_scatter(hist, [idx], ones)
        return carry

    lax.fori_loop(0, EPW // 16, body, 0)
    pltpu.sync_copy(hist, degp_hbm.at[pl.ds(wid * NP, NP)])


@functools.partial(
    pl.kernel,
    out_type=jax.ShapeDtypeStruct((NCHUNK, NP, 128), jnp.float32),
    mesh=_sc_mesh,
    scratch_types=[
        pltpu.VMEM((4, EB), jnp.int32),
        pltpu.VMEM((NB, EB), jnp.int32),
        pltpu.VMEM((2, EB, 128), jnp.float32),
        pltpu.SemaphoreType.DMA,
        pltpu.SemaphoreType.DMA,
        pltpu.SemaphoreType.DMA,
        pltpu.SemaphoreType.DMA,
        pltpu.SemaphoreType.DMA,
        pltpu.SemaphoreType.DMA,
        pltpu.VMEM_SHARED((NP, 128), jnp.float32),
    ],
    compiler_params=_sc_params,
)
def _sc_message(h4_hbm, src_hbm, dst_hbm, acc_hbm, sring, dstv, buf,
                gsem0, gsem1, is0, is1, is2, is3, acc_sh):
    cid = lax.axis_index("c")
    sid = lax.axis_index("s")
    gsems = (gsem0, gsem1)
    isems = (is0, is1, is2, is3)
    pltpu.sync_copy(dst_hbm.at[pl.ds(sid * NB, NB)], dstv)
    rs = sid * RPS
    jbase = sid * NB

    def idx_load(j, slot):
        return pltpu.async_copy(src_hbm.at[pl.ds(jbase + j, 1)],
                                sring.at[pl.ds(slot, 1)], isems[slot])

    for ch in range(NCHUNK):

        @pl.when(cid == ch // (NCHUNK // NC))
        def _():
            pltpu.sync_copy(h4_hbm.at[ch].at[pl.ds(rs, RPS)],
                            acc_sh.at[pl.ds(rs, RPS)])
            plsc.subcore_barrier()

            def gather(slot, b):
                return pltpu.async_copy(h4_hbm.at[ch].at[sring.at[slot]],
                                        buf.at[b], gsems[b])

            idx_load(0, 0)
            idx_load(1, 1)
            idx_load(2, 2)
            pltpu.make_async_copy(src_hbm.at[pl.ds(jbase, 1)],
                                  sring.at[pl.ds(0, 1)], isems[0]).wait()
            gather(0, 0)
            pltpu.make_async_copy(src_hbm.at[pl.ds(jbase, 1)],
                                  sring.at[pl.ds(1, 1)], isems[1]).wait()
            gather(1, 1)

            def phase(j, slot, b):
                nxt = (slot + 3) % 4

                @pl.when(j + 3 < NB)
                def _():
                    idx_load(j + 3, nxt)

                pltpu.make_async_copy(h4_hbm.at[ch].at[sring.at[slot]],
                                      buf.at[b], gsems[b]).wait()
                pltpu.sync_copy(buf.at[b], acc_sh.at[dstv.at[j]], add=True)

                @pl.when(j + 2 < NB)
                def _():
                    nslot = (slot + 2) % 4
                    pltpu.make_async_copy(
                        src_hbm.at[pl.ds(jbase, 1)],
                        sring.at[pl.ds(nslot, 1)], isems[nslot]).wait()
                    gather(nslot, b)

            def body(jj, carry):
                for p in range(4):
                    phase(jj * 4 + p, p, p % 2)
                return carry

            lax.fori_loop(0, NB // 4, body, 0)
            plsc.subcore_barrier()
            pltpu.sync_copy(acc_sh.at[pl.ds(rs, RPS)],
                            acc_hbm.at[ch].at[pl.ds(rs, RPS)])
            plsc.subcore_barrier()


_BN = 1024


def _to_col(dis_row):
    r = lax.broadcasted_iota(jnp.int32, (128, 128), 0)
    c = lax.broadcasted_iota(jnp.int32, (128, 128), 1)
    eye = r == c
    pieces = []
    for k in range(_BN // 128):
        sub = jnp.broadcast_to(dis_row[:, k * 128:(k + 1) * 128], (128, 128))
        pieces.append(jnp.sum(jnp.where(eye, sub, 0.0), axis=1,
                              keepdims=True))
    return jnp.concatenate(pieces, axis=0)


def _tc_layer1_body(x_ref, w_ref, degp_ref, h4_ref, dis_ref):
    deg = jnp.sum(degp_ref[...], axis=0, keepdims=True) + 1.0
    dis_row = lax.rsqrt(deg)
    dis_ref[...] = dis_row[0]
    dis = _to_col(dis_row)
    h = jnp.dot(x_ref[...], w_ref[...], preferred_element_type=jnp.float32)
    hp = h * dis
    for c in range(NCHUNK):
        h4_ref[c] = hp[:, c * 128:(c + 1) * 128]


def _tc_layer2_body(acc_ref, dis_ref, b_ref, w_ref, h4_ref):
    dis = _to_col(dis_ref[...][None, :])
    acc = acc_ref[...]
    w = w_ref[...]
    h2 = jnp.zeros((_BN, DH), jnp.float32)
    for c in range(NCHUNK):
        y = jnp.maximum(acc[c] * dis + b_ref[0, c * 128:(c + 1) * 128][None, :], 0.0)
        h2 = h2 + jnp.dot(y, w[c * 128:(c + 1) * 128, :],
                          preferred_element_type=jnp.float32)
    hp2 = h2 * dis
    for c in range(NCHUNK):
        h4_ref[c] = hp2[:, c * 128:(c + 1) * 128]


def _tc_final_body(acc_ref, dis_ref, b_ref, out_ref):
    dis = _to_col(dis_ref[...][None, :])
    acc = acc_ref[...]
    for c in range(NCHUNK):
        out_ref[:, c * 128:(c + 1) * 128] = jnp.maximum(
            acc[c] * dis + b_ref[0, c * 128:(c + 1) * 128][None, :], 0.0)


def _tc_layer1(x, W1, degp):
    return pl.pallas_call(
        _tc_layer1_body,
        grid=(NP // _BN,),
        in_specs=[
            pl.BlockSpec((_BN, D_IN), lambda i: (i, 0)),
            pl.BlockSpec((D_IN, DH), lambda i: (0, 0)),
            pl.BlockSpec((NW, _BN), lambda i: (0, i)),
        ],
        out_specs=[
            pl.BlockSpec((NCHUNK, _BN, 128), lambda i: (0, i, 0)),
            pl.BlockSpec((_BN,), lambda i: (i,)),
        ],
        out_shape=[
            jax.ShapeDtypeStruct((NCHUNK, NP, 128), jnp.float32),
            jax.ShapeDtypeStruct((NP,), jnp.float32),
        ],
    )(x, W1, degp)


def _tc_layer2(acc4, dis, b1, W2):
    return pl.pallas_call(
        _tc_layer2_body,
        grid=(NP // _BN,),
        in_specs=[
            pl.BlockSpec((NCHUNK, _BN, 128), lambda i: (0, i, 0)),
            pl.BlockSpec((_BN,), lambda i: (i,)),
            pl.BlockSpec((1, DH), lambda i: (0, 0)),
            pl.BlockSpec((DH, DH), lambda i: (0, 0)),
        ],
        out_specs=pl.BlockSpec((NCHUNK, _BN, 128), lambda i: (0, i, 0)),
        out_shape=jax.ShapeDtypeStruct((NCHUNK, NP, 128), jnp.float32),
    )(acc4, dis, b1, W2)


def _tc_final(acc4, dis, b2):
    return pl.pallas_call(
        _tc_final_body,
        grid=(NP // _BN,),
        in_specs=[
            pl.BlockSpec((NCHUNK, _BN, 128), lambda i: (0, i, 0)),
            pl.BlockSpec((_BN,), lambda i: (i,)),
            pl.BlockSpec((1, DH), lambda i: (0, 0)),
        ],
        out_specs=pl.BlockSpec((_BN, DH), lambda i: (i, 0)),
        out_shape=jax.ShapeDtypeStruct((N, DH), jnp.float32),
    )(acc4, dis, b2)


def kernel(x, edge_index, W1, b1, W2, b2):
    src = edge_index[0]
    dst = edge_index[1]
    pad = jnp.full((EP - E,), PAD_NODE, jnp.int32)
    srcp = jnp.concatenate([src, pad])
    dstp = jnp.concatenate([dst, pad])
    src2d = srcp.reshape(NS * NB, EB)
    dst2d = dstp.reshape(NS * NB, EB)

    zero_np = jnp.zeros((NP,), jnp.float32)

    degp = _sc_degree(dstp, zero_np).reshape(NW, NP)
    h4, dis = _tc_layer1(x, W1, degp)
    acc4 = _sc_message(h4, src2d, dst2d)
    h4b = _tc_layer2(acc4, dis, b1.reshape(1, DH), W2)
    acc4b = _sc_message(h4b, src2d, dst2d)
    return _tc_final(acc4b, dis, b2.reshape(1, DH))


# speedup vs baseline: 1.1045x; 1.1045x over previous
import functools

import jax
import jax.numpy as jnp
from jax import lax
from jax.experimental import pallas as pl
from jax.experimental.pallas import tpu as pltpu
from jax.experimental.pallas import tpu_sc as plsc

N = 10000
NP = 10240
D_IN = 256
DH = 512
E = 160000
EP = 163840
PAD_NODE = N
NC, NS = 2, 16
NW = NC * NS
EPW = EP // NW
EB = 128
NB = EP // NS // EB
RPS = NP // NS
NCHUNK = DH // 128

_sc_mesh = plsc.VectorSubcoreMesh(
    core_axis_name="c", subcore_axis_name="s", num_cores=NC, num_subcores=NS)
_sc_params = pltpu.CompilerParams(needs_layout_passes=False)


@functools.partial(
    pl.kernel,
    out_type=jax.ShapeDtypeStruct((NW * NP,), jnp.float32),
    mesh=_sc_mesh,
    scratch_types=[
        pltpu.VMEM((EPW,), jnp.int32),
        pltpu.VMEM((NP,), jnp.float32),
    ],
    compiler_params=_sc_params,
)
def _sc_degree(dst_hbm, zero_hbm, degp_hbm, dstv, hist):
    cid = lax.axis_index("c")
    sid = lax.axis_index("s")
    wid = sid * NC + cid
    pltpu.sync_copy(zero_hbm, hist)
    pltpu.sync_copy(dst_hbm.at[pl.ds(wid * EPW, EPW)], dstv)
    ones = jnp.ones((16,), jnp.float32)

    def body(j, carry):
        idx = dstv[pl.ds(j * 16, 16)]
        plsc.addupdate_scatter(hist, [idx], ones)
        return carry

    lax.fori_loop(0, EPW // 16, body, 0)
    pltpu.sync_copy(hist, degp_hbm.at[pl.ds(wid * NP, NP)])


@functools.partial(
    pl.kernel,
    out_type=jax.ShapeDtypeStruct((NCHUNK, NP, 128), jnp.float32),
    mesh=_sc_mesh,
    scratch_types=[
        pltpu.VMEM((4, EB), jnp.int32),
        pltpu.VMEM((NB, EB), jnp.int32),
        pltpu.VMEM((2, EB, 128), jnp.float32),
        pltpu.SemaphoreType.DMA,
        pltpu.SemaphoreType.DMA,
        pltpu.SemaphoreType.DMA,
        pltpu.SemaphoreType.DMA,
        pltpu.SemaphoreType.DMA,
        pltpu.SemaphoreType.DMA,
        pltpu.VMEM_SHARED((NP, 128), jnp.float32),
    ],
    compiler_params=_sc_params,
)
def _sc_message(h4_hbm, src_hbm, dst_hbm, acc_hbm, sring, dstv, buf,
                gsem0, gsem1, is0, is1, is2, is3, acc_sh):
    cid = lax.axis_index("c")
    sid = lax.axis_index("s")
    gsems = (gsem0, gsem1)
    isems = (is0, is1, is2, is3)
    pltpu.sync_copy(dst_hbm.at[pl.ds(sid * NB, NB)], dstv)
    rs = sid * RPS
    jbase = sid * NB

    def idx_load(j, slot):
        return pltpu.async_copy(src_hbm.at[pl.ds(jbase + j, 1)],
                                sring.at[pl.ds(slot, 1)], isems[slot])

    for ch in range(NCHUNK):

        @pl.when(cid == ch // (NCHUNK // NC))
        def _():
            pltpu.sync_copy(h4_hbm.at[ch].at[pl.ds(rs, RPS)],
                            acc_sh.at[pl.ds(rs, RPS)])
            plsc.subcore_barrier()

            def gather(slot, b):
                return pltpu.async_copy(h4_hbm.at[ch].at[sring.at[slot]],
                                        buf.at[b], gsems[b])

            idx_load(0, 0)
            idx_load(1, 1)
            idx_load(2, 2)
            pltpu.make_async_copy(src_hbm.at[pl.ds(jbase, 1)],
                                  sring.at[pl.ds(0, 1)], isems[0]).wait()
            gather(0, 0)
            pltpu.make_async_copy(src_hbm.at[pl.ds(jbase, 1)],
                                  sring.at[pl.ds(1, 1)], isems[1]).wait()
            gather(1, 1)

            def phase(j, slot, b):
                nxt = (slot + 3) % 4

                @pl.when(j + 3 < NB)
                def _():
                    idx_load(j + 3, nxt)

                pltpu.make_async_copy(h4_hbm.at[ch].at[sring.at[slot]],
                                      buf.at[b], gsems[b]).wait()
                pltpu.sync_copy(buf.at[b], acc_sh.at[dstv.at[j]], add=True)

                @pl.when(j + 2 < NB)
                def _():
                    nslot = (slot + 2) % 4
                    pltpu.make_async_copy(
                        src_hbm.at[pl.ds(jbase, 1)],
                        sring.at[pl.ds(nslot, 1)], isems[nslot]).wait()
                    gather(nslot, b)

            def body(jj, carry):
                for p in range(4):
                    phase(jj * 4 + p, p, p % 2)
                return carry

            lax.fori_loop(0, NB // 4, body, 0)
            plsc.subcore_barrier()
            pltpu.sync_copy(acc_sh.at[pl.ds(rs, RPS)],
                            acc_hbm.at[ch].at[pl.ds(rs, RPS)])
            plsc.subcore_barrier()


_BN = 1024


def _to_col(dis_row):
    r = lax.broadcasted_iota(jnp.int32, (128, 128), 0)
    c = lax.broadcasted_iota(jnp.int32, (128, 128), 1)
    eye = r == c
    pieces = []
    for k in range(_BN // 128):
        sub = jnp.broadcast_to(dis_row[:, k * 128:(k + 1) * 128], (128, 128))
        pieces.append(jnp.sum(jnp.where(eye, sub, 0.0), axis=1,
                              keepdims=True))
    return jnp.concatenate(pieces, axis=0)


def _tc_layer1_body(x_ref, w_ref, degp_ref, h4_ref, dis_ref):
    deg = jnp.sum(degp_ref[...], axis=0, keepdims=True) + 1.0
    dis_row = lax.rsqrt(deg)
    dis_ref[...] = dis_row[0]
    dis = _to_col(dis_row)
    h = jnp.dot(x_ref[...], w_ref[...], preferred_element_type=jnp.float32)
    hp = h * dis
    for c in range(NCHUNK):
        h4_ref[c] = hp[:, c * 128:(c + 1) * 128]


def _tc_layer2_body(acc_ref, dis_ref, b_ref, w_ref, h4_ref):
    dis = _to_col(dis_ref[...][None, :])
    acc = acc_ref[...]
    w = w_ref[...]
    h2 = jnp.zeros((_BN, DH), jnp.float32)
    for c in range(NCHUNK):
        y = jnp.maximum(acc[c] * dis + b_ref[0, c * 128:(c + 1) * 128][None, :], 0.0)
        h2 = h2 + jnp.dot(y, w[c * 128:(c + 1) * 128, :],
                          preferred_element_type=jnp.float32)
    hp2 = h2 * dis
    for c in range(NCHUNK):
        h4_ref[c] = hp2[:, c * 128:(c + 1) * 128]


def _tc_final_body(acc_ref, dis_ref, b_ref, out_ref):
    dis = _to_col(dis_ref[...][None, :])
    acc = acc_ref[...]
    for c in range(NCHUNK):
        out_ref[:, c * 128:(c + 1) * 128] = jnp.maximum(
            acc[c] * dis + b_ref[0, c * 128:(c + 1) * 128][None, :], 0.0)


def _tc_layer1(x_pad, W1, degp):
    return pl.pallas_call(
        _tc_layer1_body,
        grid=(NP // _BN,),
        in_specs=[
            pl.BlockSpec((_BN, D_IN), lambda i: (i, 0)),
            pl.BlockSpec((D_IN, DH), lambda i: (0, 0)),
            pl.BlockSpec((NW, _BN), lambda i: (0, i)),
        ],
        out_specs=[
            pl.BlockSpec((NCHUNK, _BN, 128), lambda i: (0, i, 0)),
            pl.BlockSpec((_BN,), lambda i: (i,)),
        ],
        out_shape=[
            jax.ShapeDtypeStruct((NCHUNK, NP, 128), jnp.float32),
            jax.ShapeDtypeStruct((NP,), jnp.float32),
        ],
    )(x_pad, W1, degp)


def _tc_layer2(acc4, dis, b1, W2):
    return pl.pallas_call(
        _tc_layer2_body,
        grid=(NP // _BN,),
        in_specs=[
            pl.BlockSpec((NCHUNK, _BN, 128), lambda i: (0, i, 0)),
            pl.BlockSpec((_BN,), lambda i: (i,)),
            pl.BlockSpec((1, DH), lambda i: (0, 0)),
            pl.BlockSpec((DH, DH), lambda i: (0, 0)),
        ],
        out_specs=pl.BlockSpec((NCHUNK, _BN, 128), lambda i: (0, i, 0)),
        out_shape=jax.ShapeDtypeStruct((NCHUNK, NP, 128), jnp.float32),
    )(acc4, dis, b1, W2)


def _tc_final(acc4, dis, b2):
    return pl.pallas_call(
        _tc_final_body,
        grid=(NP // _BN,),
        in_specs=[
            pl.BlockSpec((NCHUNK, _BN, 128), lambda i: (0, i, 0)),
            pl.BlockSpec((_BN,), lambda i: (i,)),
            pl.BlockSpec((1, DH), lambda i: (0, 0)),
        ],
        out_specs=pl.BlockSpec((_BN, DH), lambda i: (i, 0)),
        out_shape=jax.ShapeDtypeStruct((NP, DH), jnp.float32),
    )(acc4, dis, b2)


def kernel(x, edge_index, W1, b1, W2, b2):
    src = edge_index[0]
    dst = edge_index[1]
    pad = jnp.full((EP - E,), PAD_NODE, jnp.int32)
    srcp = jnp.concatenate([src, pad])
    dstp = jnp.concatenate([dst, pad])
    src2d = srcp.reshape(NS * NB, EB)
    dst2d = dstp.reshape(NS * NB, EB)

    x_pad = jnp.zeros((NP, D_IN), jnp.float32).at[:N].set(x)
    zero_np = jnp.zeros((NP,), jnp.float32)

    degp = _sc_degree(dstp, zero_np).reshape(NW, NP)
    h4, dis = _tc_layer1(x_pad, W1, degp)
    acc4 = _sc_message(h4, src2d, dst2d)
    h4b = _tc_layer2(acc4, dis, b1.reshape(1, DH), W2)
    acc4b = _sc_message(h4b, src2d, dst2d)
    out = _tc_final(acc4b, dis, b2.reshape(1, DH))
    return out[:N]
